# baseline (speedup 1.0000x reference)
import jax
import jax.numpy as jnp
from jax import lax
from jax.experimental import pallas as pl
from jax.experimental.pallas import tpu as pltpu

_N = 16384
_C = 1000
_BR = 512
_NB = _N // _BR


def _nll_block(prob_hbm, tgt_ref, w_ref, out_ref, buf, sem):
    i = pl.program_id(0)
    slot = lax.rem(i, 2)
    nslot = 1 - slot

    @pl.when(i == 0)
    def _prologue():
        pltpu.make_async_copy(
            prob_hbm.at[pl.ds(0, _BR), :], buf.at[0], sem.at[0]).start()

    @pl.when(i + 1 < _NB)
    def _prefetch():
        pltpu.make_async_copy(
            prob_hbm.at[pl.ds((i + 1) * _BR, _BR), :],
            buf.at[nslot], sem.at[nslot]).start()

    pltpu.make_async_copy(
        prob_hbm.at[pl.ds(i * _BR, _BR), :], buf.at[slot], sem.at[slot]).wait()

    t = tgt_ref[0, :, :]
    col = lax.broadcasted_iota(jnp.int32, (_BR, _C), 1)
    masked = jnp.where(col == t, buf[slot], 0.0)
    s = jnp.sum(masked, axis=0, keepdims=True)
    out_ref[...] = jnp.sum(s * w_ref[...]).reshape(1, 1, 1)


def kernel(prob, target, weight):
    tgt_3d = target.reshape(_NB, _BR, 1)
    partials = pl.pallas_call(
        _nll_block,
        grid=(_NB,),
        in_specs=[
            pl.BlockSpec(memory_space=pl.ANY),
            pl.BlockSpec((1, _BR, 1), lambda i: (i, 0, 0)),
            pl.BlockSpec((1, _C), lambda i: (0, 0)),
        ],
        out_specs=pl.BlockSpec((1, 1, 1), lambda i: (i, 0, 0)),
        out_shape=jax.ShapeDtypeStruct((_NB, 1, 1), jnp.float32),
        compiler_params=pltpu.CompilerParams(needs_layout_passes=False),
        scratch_shapes=[
            pltpu.VMEM((2, _BR, _C), jnp.float32),
            pltpu.SemaphoreType.DMA((2,)),
        ],
    )(prob, tgt_3d, weight.reshape(1, _C))
    return -jnp.sum(partials)


# speedup vs baseline: 4.1454x; 4.1454x over previous
import jax
import jax.numpy as jnp
from jax import lax
from jax.experimental import pallas as pl

_N = 16384
_C = 1000
_BN = 2048
_NB = _N // _BN


def _nll_block(probt_ref, tgt_ref, w_ref, out_ref):
    t = tgt_ref[0, 0, :]
    crow = lax.broadcasted_iota(jnp.int32, (_C, _BN), 0)
    masked = jnp.where(crow == t[None, :], probt_ref[...], 0.0)
    s = jnp.sum(masked, axis=1, keepdims=True)
    out_ref[...] = jnp.sum(s * w_ref[...]).reshape(1, 1, 1)


_nll_partials = pl.pallas_call(
    _nll_block,
    grid=(_NB,),
    in_specs=[
        pl.BlockSpec((_C, _BN), lambda i: (0, i)),
        pl.BlockSpec((1, 1, _BN), lambda i: (i, 0, 0)),
        pl.BlockSpec((_C, 1), lambda i: (0, 0)),
    ],
    out_specs=pl.BlockSpec((1, 1, 1), lambda i: (i, 0, 0)),
    out_shape=jax.ShapeDtypeStruct((_NB, 1, 1), jnp.float32),
)


def kernel(prob, target, weight):
    tgt_3d = target.reshape(_NB, 1, _BN)
    partials = _nll_partials(prob.T, tgt_3d, weight.reshape(_C, 1))
    return -jnp.sum(partials)
